# Initial kernel scaffold; baseline (speedup 1.0000x reference)
import jax
import jax.numpy as jnp
from jax.experimental import pallas as pl


def kernel(x, W1, b1, g1, be1, coeffs, mcb, g2, be2, W3, b3, g3, be3, Ws, bs, gs, bes, g_rows, g_cols, g_vals, l_rows, l_cols, l_vals, f_rows, f_cols, f_vals, ns, ew, vert_idx, patches):
    raise NotImplementedError("write your pallas kernel here")


# speedup vs baseline: 1.2114x; 1.2114x over previous
import functools
import jax
import jax.numpy as jnp
from jax.experimental import pallas as pl
from jax.experimental.pallas import tpu as pltpu

B = 8
IN_CH = 256
OUT_CH = 512
NV_FINE = 10242
NV_COARSE = 2562
NF = 5120
N_ROWS = B * NV_COARSE
ROW_TILE = 168
N_ROW_TILES = N_ROWS // ROW_TILE
VF_TILE = 512
N_VF_TILES = (NV_FINE + VF_TILE - 1) // VF_TILE
VC_TILE = 128
N_VC_TILES = (NV_COARSE + VC_TILE - 1) // VC_TILE
EPS = 1e-5


def _gelu(x):
    return 0.5 * x * (1.0 + jax.lax.erf(x * 0.7071067811865476))


def _conv_fine_body(x_ref, w_ref, y_ref):
    xb = x_ref[0]
    y_ref[0] = jax.lax.dot_general(
        xb, w_ref[...], (((0,), (0,)), ((), ())),
        preferred_element_type=jnp.float32)


def _conv_fine(x, wcat_t):
    return pl.pallas_call(
        _conv_fine_body,
        grid=(B, N_VF_TILES),
        in_specs=[
            pl.BlockSpec((1, IN_CH, VF_TILE), lambda b, i: (b, 0, i)),
            pl.BlockSpec((IN_CH, IN_CH + OUT_CH), lambda b, i: (0, 0)),
        ],
        out_specs=pl.BlockSpec((1, VF_TILE, IN_CH + OUT_CH), lambda b, i: (b, i, 0)),
        out_shape=jax.ShapeDtypeStruct((B, NV_FINE, IN_CH + OUT_CH), jnp.float32),
    )(x, wcat_t)


def _stats_body(x_ref, o_ref):
    i = pl.program_id(0)

    @pl.when(i == 0)
    def _():
        o_ref[...] = jnp.zeros_like(o_ref)

    r = x_ref[...]
    o_ref[...] += jnp.stack([jnp.sum(r, axis=0), jnp.sum(r * r, axis=0)])


def _stats(rows):
    c = rows.shape[1]
    return pl.pallas_call(
        _stats_body,
        grid=(N_ROW_TILES,),
        in_specs=[pl.BlockSpec((ROW_TILE, c), lambda i: (i, 0))],
        out_specs=pl.BlockSpec((2, c), lambda i: (0, 0)),
        out_shape=jax.ShapeDtypeStruct((2, c), jnp.float32),
    )(rows)


def _scale_off(sums, g, be):
    mean = sums[0] / N_ROWS
    var = sums[1] / N_ROWS - mean * mean
    scale = g * jax.lax.rsqrt(var + EPS)
    return scale, be - mean * scale


def _apply_gelu_body(x_ref, s_ref, g_ref, b_ref, o_ref):
    scale, off = _scale_off(s_ref[...], g_ref[...], b_ref[...])
    o_ref[...] = _gelu(x_ref[...] * scale[None, :] + off[None, :])


def _apply_gelu(rows, sums, g, be):
    c = rows.shape[1]
    return pl.pallas_call(
        _apply_gelu_body,
        grid=(N_ROW_TILES,),
        in_specs=[
            pl.BlockSpec((ROW_TILE, c), lambda i: (i, 0)),
            pl.BlockSpec((2, c), lambda i: (0, 0)),
            pl.BlockSpec((c,), lambda i: (0,)),
            pl.BlockSpec((c,), lambda i: (0,)),
        ],
        out_specs=pl.BlockSpec((ROW_TILE, c), lambda i: (i, 0)),
        out_shape=jax.ShapeDtypeStruct((N_ROWS, c), jnp.float32),
    )(rows, sums, g, be)


def _feat_mm_body(h_ref, lap_ref, gve_ref, gvn_ref, w_ref, o_ref):
    ht = h_ref[...].reshape(VC_TILE * B, IN_CH)
    lt = lap_ref[...].reshape(VC_TILE * B, IN_CH)
    et = gve_ref[...].reshape(VC_TILE * B, IN_CH)
    nt = gvn_ref[...].reshape(VC_TILE * B, IN_CH)
    w = w_ref[...]
    acc = jnp.dot(ht, w[0:IN_CH], preferred_element_type=jnp.float32)
    acc += jnp.dot(lt, w[IN_CH:2 * IN_CH], preferred_element_type=jnp.float32)
    acc += jnp.dot(et, w[2 * IN_CH:3 * IN_CH], preferred_element_type=jnp.float32)
    acc += jnp.dot(nt, w[3 * IN_CH:4 * IN_CH], preferred_element_type=jnp.float32)
    o_ref[...] = acc.reshape(VC_TILE, B, IN_CH)


def _feat_mm(h3, lap3, gve3, gvn3, wm):
    spec = pl.BlockSpec((VC_TILE, B, IN_CH), lambda i: (i, 0, 0))
    return pl.pallas_call(
        _feat_mm_body,
        grid=(N_VC_TILES,),
        in_specs=[spec, spec, spec, spec,
                  pl.BlockSpec((4 * IN_CH, IN_CH), lambda i: (0, 0))],
        out_specs=spec,
        out_shape=jax.ShapeDtypeStruct((NV_COARSE, B, IN_CH), jnp.float32),
    )(h3, lap3, gve3, gvn3, wm)


def _w3_body(m_ref, s_ref, g_ref, b_ref, w_ref, o_ref):
    scale, off = _scale_off(s_ref[...], g_ref[...], b_ref[...])
    z = _gelu(m_ref[...] * scale[None, None, :] + off[None, None, :])
    t = jnp.dot(z.reshape(VC_TILE * B, IN_CH), w_ref[...],
                preferred_element_type=jnp.float32)
    o_ref[...] = jnp.transpose(t.reshape(VC_TILE, B, OUT_CH), (1, 0, 2))


def _w3(m3, sums2, g2, be2, w3t):
    return pl.pallas_call(
        _w3_body,
        grid=(N_VC_TILES,),
        in_specs=[
            pl.BlockSpec((VC_TILE, B, IN_CH), lambda i: (i, 0, 0)),
            pl.BlockSpec((2, IN_CH), lambda i: (0, 0)),
            pl.BlockSpec((IN_CH,), lambda i: (0,)),
            pl.BlockSpec((IN_CH,), lambda i: (0,)),
            pl.BlockSpec((IN_CH, OUT_CH), lambda i: (0, 0)),
        ],
        out_specs=pl.BlockSpec((B, VC_TILE, OUT_CH), lambda i: (0, i, 0)),
        out_shape=jax.ShapeDtypeStruct((B, NV_COARSE, OUT_CH), jnp.float32),
    )(m3, sums2, g2, be2, w3t)


def _final_body(t_ref, p_ref, s3_ref, g3_ref, b3_ref, ss_ref, gs_ref, bs_ref, o_ref):
    sc3, of3 = _scale_off(s3_ref[...], g3_ref[...], b3_ref[...])
    scs, ofs = _scale_off(ss_ref[...], gs_ref[...], bs_ref[...])
    r = (t_ref[0] * sc3[None, :] + of3[None, :]
         + p_ref[0] * scs[None, :] + ofs[None, :])
    r = _gelu(r)
    o_ref[0] = jnp.transpose(r, (1, 0))


def _final(t, ps, sums3, g3, be3, sums_s, gs, bes):
    vec = pl.BlockSpec((OUT_CH,), lambda b, i: (0,))
    st = pl.BlockSpec((2, OUT_CH), lambda b, i: (0, 0))
    blk = pl.BlockSpec((1, VC_TILE, OUT_CH), lambda b, i: (b, i, 0))
    return pl.pallas_call(
        _final_body,
        grid=(B, N_VC_TILES),
        in_specs=[blk, blk, st, vec, vec, st, vec, vec],
        out_specs=pl.BlockSpec((1, OUT_CH, VC_TILE), lambda b, i: (b, 0, i)),
        out_shape=jax.ShapeDtypeStruct((B, OUT_CH, NV_COARSE), jnp.float32),
    )(t, ps, sums3, g3, be3, sums_s, gs, bes)


def kernel(x, W1, b1, g1, be1, coeffs, mcb, g2, be2, W3, b3, g3, be3,
           Ws, bs, gs, bes, g_rows, g_cols, g_vals, l_rows, l_cols, l_vals,
           f_rows, f_cols, f_vals, ns, ew, vert_idx, patches):
    wcat_t = jnp.concatenate([W1, Ws], axis=0).T
    wm = jnp.transpose(coeffs, (2, 1, 0)).reshape(4 * IN_CH, IN_CH)
    w3t = W3.T
    pidx = vert_idx[patches].astype(jnp.int32)
    gc9 = jnp.transpose(g_cols.astype(jnp.int32).reshape(3, NF, 3),
                        (1, 0, 2)).reshape(NF, 9)
    gv9 = jnp.transpose(g_vals.reshape(3, NF, 3), (1, 0, 2)).reshape(NF, 3, 3)
    we9 = (ew[:, :, None] * gv9).reshape(NF, 9)
    wn9 = (ns[:, :, None] * gv9).reshape(NF, 9)
    lc7 = l_cols.astype(jnp.int32).reshape(NV_COARSE, 7)
    lv7 = l_vals.reshape(NV_COARSE, 7)
    fc6 = f_cols.astype(jnp.int32).reshape(NV_COARSE, 6)
    fv6 = f_vals.reshape(NV_COARSE, 6)

    y = _conv_fine(x, wcat_t)

    pooled = jnp.max(y[:, pidx, :], axis=2)
    h_rows = jnp.transpose(pooled[:, :, :IN_CH], (1, 0, 2)).reshape(N_ROWS, IN_CH)
    ps = pooled[:, :, IN_CH:]

    sums1 = _stats(h_rows)
    h = _apply_gelu(h_rows, sums1, g1, be1)
    h3 = h.reshape(NV_COARSE, B, IN_CH)
    hv = h.reshape(NV_COARSE, B * IN_CH)

    lap = jnp.einsum('vk,vkc->vc', lv7, hv[lc7]).reshape(NV_COARSE, B, IN_CH)
    gfe = jnp.einsum('fk,fkc->fc', we9, hv[gc9])
    gfn = jnp.einsum('fk,fkc->fc', wn9, hv[gc9])
    gve = jnp.einsum('vk,vkc->vc', fv6, gfe[fc6]).reshape(NV_COARSE, B, IN_CH)
    gvn = jnp.einsum('vk,vkc->vc', fv6, gfn[fc6]).reshape(NV_COARSE, B, IN_CH)

    m3 = _feat_mm(h3, lap, gve, gvn, wm)

    sums2 = _stats(m3.reshape(N_ROWS, IN_CH))
    t = _w3(m3, sums2, g2, be2, w3t)

    sums3 = _stats(t.reshape(N_ROWS, OUT_CH))
    sums_s = _stats(ps.reshape(N_ROWS, OUT_CH))
    return _final(t, ps, sums3, g3, be3, sums_s, gs, bes)
